# baseline (speedup 1.0000x reference)
import functools

import jax
import jax.numpy as jnp
from jax import lax
from jax.experimental import pallas as pl
from jax.experimental.pallas import tpu as pltpu
from jax.experimental.pallas import tpu_sc as plsc

R = 8192
D = 768
NC = 2
NS = 16
NW = NC * NS
ROWS_PER_W = R // NW
CH = 16
N_CHUNKS = ROWS_PER_W // CH
NBUF = 10
RA = 6


@functools.partial(
    pl.kernel,
    out_type=jax.ShapeDtypeStruct((R, D), jnp.float32),
    mesh=plsc.VectorSubcoreMesh(core_axis_name="c", subcore_axis_name="s"),
    scratch_types=(
        [pltpu.VMEM((16,), jnp.int32),
         pltpu.VMEM((1, D), jnp.float32),
         pltpu.VMEM_SHARED((NS * NBUF * CH, D), jnp.float32)]
        + [pltpu.SemaphoreType.DMA for _ in range(2 * NBUF)]
    ),
)
def _emb_lookup(t_hbm, w_hbm, out_hbm, tvec_v, rowbuf_v, spm, *sems):
    rsems = sems[:NBUF]
    wsems = sems[NBUF:]
    sid = lax.axis_index("s")
    wid = sid * NC + lax.axis_index("c")
    base = wid * ROWS_PER_W
    bufs = [spm.at[pl.ds((sid * NBUF + b) * CH, CH)] for b in range(NBUF)]

    reads = [None] * N_CHUNKS
    writes = [None] * N_CHUNKS
    for c in range(min(RA, N_CHUNKS)):
        reads[c] = pltpu.async_copy(
            w_hbm.at[pl.ds(base + c * CH, CH)], bufs[c % NBUF], rsems[c % NBUF])

    pltpu.sync_copy(t_hbm, tvec_v.at[pl.ds(0, 1)])
    t = tvec_v[...][0]
    tl = jnp.clip(t, 1, R)

    for c in range(N_CHUNKS):
        b = c % NBUF
        reads[c].wait()
        writes[c] = pltpu.async_copy(
            bufs[b], out_hbm.at[pl.ds(base + c * CH, CH)], wsems[b])
        nxt = c + RA
        if nxt < N_CHUNKS:
            nb = nxt % NBUF
            prev = nxt - NBUF
            if prev >= 0:
                writes[prev].wait()
            reads[nxt] = pltpu.async_copy(
                w_hbm.at[pl.ds(base + nxt * CH, CH)], bufs[nb], rsems[nb])
    for c in range(max(0, N_CHUNKS - NBUF), N_CHUNKS):
        writes[c].wait()

    pltpu.sync_copy(w_hbm.at[pl.ds(tl - 1, 1)], rowbuf_v)
    start = jnp.maximum(tl, base)

    def _fix(i, carry):
        pltpu.sync_copy(rowbuf_v, out_hbm.at[pl.ds(i, 1)])
        return carry

    lax.fori_loop(start, base + ROWS_PER_W, _fix, 0)


def kernel(T, weight):
    t_arr = jnp.asarray(T, jnp.int32).reshape(1)
    return _emb_lookup(t_arr, weight)


# speedup vs baseline: 1.0765x; 1.0765x over previous
import functools

import jax
import jax.numpy as jnp
from jax import lax
from jax.experimental import pallas as pl
from jax.experimental.pallas import tpu as pltpu
from jax.experimental.pallas import tpu_sc as plsc

R = 8192
D = 768
NC = 2
NS = 16
NW = NC * NS
ROWS_PER_W = R // NW
CH = 16
N_CHUNKS = ROWS_PER_W // CH
NBUF = 10
RA = 6


@functools.partial(
    pl.kernel,
    out_type=jax.ShapeDtypeStruct((R, D), jnp.float32),
    mesh=plsc.VectorSubcoreMesh(core_axis_name="c", subcore_axis_name="s"),
    scratch_types=(
        [pltpu.VMEM((16,), jnp.int32),
         pltpu.VMEM((1, D), jnp.float32)]
        + [pltpu.VMEM((CH, D), jnp.float32) for _ in range(NBUF)]
        + [pltpu.SemaphoreType.DMA for _ in range(2 * NBUF)]
    ),
)
def _emb_lookup(t_hbm, w_hbm, out_hbm, tvec_v, rowbuf_v, *bufs_and_sems):
    bufs = bufs_and_sems[:NBUF]
    rsems = bufs_and_sems[NBUF:2 * NBUF]
    wsems = bufs_and_sems[2 * NBUF:]
    wid = lax.axis_index("s") * NC + lax.axis_index("c")
    base = wid * ROWS_PER_W

    reads = [None] * N_CHUNKS
    writes = [None] * N_CHUNKS
    for c in range(min(RA, N_CHUNKS)):
        reads[c] = pltpu.async_copy(
            w_hbm.at[pl.ds(base + c * CH, CH)], bufs[c % NBUF], rsems[c % NBUF])

    pltpu.sync_copy(t_hbm, tvec_v.at[pl.ds(0, 1)])
    t = tvec_v[...][0]
    tl = jnp.clip(t, 1, R)

    for c in range(N_CHUNKS):
        b = c % NBUF
        reads[c].wait()
        writes[c] = pltpu.async_copy(
            bufs[b], out_hbm.at[pl.ds(base + c * CH, CH)], wsems[b])
        nxt = c + RA
        if nxt < N_CHUNKS:
            nb = nxt % NBUF
            prev = nxt - NBUF
            if prev >= 0:
                writes[prev].wait()
            reads[nxt] = pltpu.async_copy(
                w_hbm.at[pl.ds(base + nxt * CH, CH)], bufs[nb], rsems[nb])
    for c in range(max(0, N_CHUNKS - NBUF), N_CHUNKS):
        writes[c].wait()

    start = jnp.maximum(tl, base)

    @pl.when(start < base + ROWS_PER_W)
    def _patch():
        pltpu.sync_copy(w_hbm.at[pl.ds(tl - 1, 1)], rowbuf_v)

        def _fix(i, carry):
            pltpu.sync_copy(rowbuf_v, out_hbm.at[pl.ds(i, 1)])
            return carry

        lax.fori_loop(start, base + ROWS_PER_W, _fix, 0)


def kernel(T, weight):
    t_arr = jnp.asarray(T, jnp.int32).reshape(1)
    return _emb_lookup(t_arr, weight)


# speedup vs baseline: 1.0836x; 1.0066x over previous
import functools

import jax
import jax.numpy as jnp
from jax import lax
from jax.experimental import pallas as pl
from jax.experimental.pallas import tpu as pltpu
from jax.experimental.pallas import tpu_sc as plsc

R = 8192
D = 768
NC = 2
NS = 16
NW = NC * NS
ROWS_PER_W = R // NW
CH = 16
N_CHUNKS = ROWS_PER_W // CH
NBUF = 10
RA = 5


@functools.partial(
    pl.kernel,
    out_type=jax.ShapeDtypeStruct((R, D), jnp.float32),
    mesh=plsc.VectorSubcoreMesh(core_axis_name="c", subcore_axis_name="s"),
    scratch_types=(
        [pltpu.VMEM((16,), jnp.int32),
         pltpu.VMEM((1, D), jnp.float32)]
        + [pltpu.VMEM((CH, D), jnp.float32) for _ in range(NBUF)]
        + [pltpu.SemaphoreType.DMA for _ in range(2 * NBUF)]
    ),
)
def _emb_lookup(t_hbm, w_hbm, out_hbm, tvec_v, rowbuf_v, *bufs_and_sems):
    bufs = bufs_and_sems[:NBUF]
    rsems = bufs_and_sems[NBUF:2 * NBUF]
    wsems = bufs_and_sems[2 * NBUF:]
    wid = lax.axis_index("s") * NC + lax.axis_index("c")
    base = wid * ROWS_PER_W

    reads = [None] * N_CHUNKS
    writes = [None] * N_CHUNKS
    for c in range(min(RA, N_CHUNKS)):
        reads[c] = pltpu.async_copy(
            w_hbm.at[pl.ds(base + c * CH, CH)], bufs[c % NBUF], rsems[c % NBUF])

    pltpu.sync_copy(t_hbm, tvec_v.at[pl.ds(0, 1)])
    t = tvec_v[...][0]
    tl = jnp.clip(t, 1, R)

    for c in range(N_CHUNKS):
        b = c % NBUF
        reads[c].wait()
        writes[c] = pltpu.async_copy(
            bufs[b], out_hbm.at[pl.ds(base + c * CH, CH)], wsems[b])
        nxt = c + RA
        if nxt < N_CHUNKS:
            nb = nxt % NBUF
            prev = nxt - NBUF
            if prev >= 0:
                writes[prev].wait()
            reads[nxt] = pltpu.async_copy(
                w_hbm.at[pl.ds(base + nxt * CH, CH)], bufs[nb], rsems[nb])
    for c in range(max(0, N_CHUNKS - NBUF), N_CHUNKS):
        writes[c].wait()

    start = jnp.maximum(tl, base)

    @pl.when(start < base + ROWS_PER_W)
    def _patch():
        pltpu.sync_copy(w_hbm.at[pl.ds(tl - 1, 1)], rowbuf_v)

        def _fix(i, carry):
            pltpu.sync_copy(rowbuf_v, out_hbm.at[pl.ds(i, 1)])
            return carry

        lax.fori_loop(start, base + ROWS_PER_W, _fix, 0)


def kernel(T, weight):
    t_arr = jnp.asarray(T, jnp.int32).reshape(1)
    return _emb_lookup(t_arr, weight)
